# baseline (speedup 1.0000x reference)
import functools

import jax
import jax.numpy as jnp
from jax.experimental import pallas as pl

SH_DIM = 16
NUM_ATOMS = 512
B, N = 2048, 128
BB = 8

C0 = 0.28209479177387814
C1 = 0.4886025119029199
C2 = (1.0925484305920792, -1.0925484305920792, 0.31539156525252005,
      -1.0925484305920792, 0.5462742152960396)
C3 = (-0.5900435899266435, 2.890611442640554, -0.4570457994644658,
      0.3731763325901154, -0.4570457994644658, 1.445305721320277,
      -0.5900435899266435)


def _sh_basis(x, y, z):
    xx, yy, zz = x * x, y * y, z * z
    xy, yz, xz = x * y, y * z, x * z
    cols = [
        C0 * jnp.ones_like(x),
        -C1 * y,
        C1 * z,
        -C1 * x,
        C2[0] * xy,
        C2[1] * yz,
        C2[2] * (2.0 * zz - xx - yy),
        C2[3] * xz,
        C2[4] * (xx - yy),
        C3[0] * y * (3.0 * xx - yy),
        C3[1] * xy * z,
        C3[2] * y * (4.0 * zz - xx - yy),
        C3[3] * z * (2.0 * zz - 3.0 * xx - 3.0 * yy),
        C3[4] * x * (4.0 * zz - xx - yy),
        C3[5] * z * (xx - yy),
        C3[6] * x * (xx - 3.0 * yy),
    ]
    return jnp.concatenate(cols, axis=1)


def _render_block(o_ref, d_ref, t0_ref, t1_ref, mask_ref, atomsT_ref,
                  wg_ref, alpha_ref, pack_ref):
    f32 = jnp.float32
    o = o_ref[...]
    d = d_ref[...]
    ox = o[:, 0].reshape(BB, 1)
    oy = o[:, 1].reshape(BB, 1)
    oz = o[:, 2].reshape(BB, 1)
    dx = d[:, 0].reshape(BB, 1)
    dy = d[:, 1].reshape(BB, 1)
    dz = d[:, 2].reshape(BB, 1)

    bf16 = jnp.bfloat16
    wgr = wg_ref[...].astype(bf16).astype(f32)
    w0 = wgr[0, :].reshape(1, 1, NUM_ATOMS)
    w1 = wgr[1, :].reshape(1, 1, NUM_ATOMS)
    w2 = wgr[2, :].reshape(1, 1, NUM_ATOMS)

    t0 = t0_ref[...]
    t1 = t1_ref[...]
    t_mid = 0.5 * (t0 + t1)

    dnorm2 = dx * dx + dy * dy + dz * dz
    dnorm = jnp.sqrt(dnorm2)
    inv_dn = 1.0 / (dnorm + 1e-12)

    sh = _sh_basis(dx * inv_dn, dy * inv_dn, dz * inv_dn)
    atomsT = atomsT_ref[...].astype(bf16).astype(f32)
    proj_r = jnp.dot(sh, atomsT[0:16, :], preferred_element_type=f32, precision=jax.lax.Precision.HIGHEST)
    proj_g = jnp.dot(sh, atomsT[16:32, :], preferred_element_type=f32, precision=jax.lax.Precision.HIGHEST)
    proj_b = jnp.dot(sh, atomsT[32:48, :], preferred_element_type=f32, precision=jax.lax.Precision.HIGHEST)
    sig_col = atomsT[48, :].reshape(1, 1, NUM_ATOMS)

    px = (ox + t_mid * dx).astype(bf16).astype(f32)[:, :, None]
    py = (oy + t_mid * dy).astype(bf16).astype(f32)[:, :, None]
    pz = (oz + t_mid * dz).astype(bf16).astype(f32)[:, :, None]
    logits = px * w0 + py * w1 + pz * w2

    m = jnp.max(logits, axis=-1, keepdims=True)
    u = jnp.exp(logits - m)
    z = jnp.sum(u, axis=-1, keepdims=True)
    q = (u / z).astype(bf16).astype(f32)
    sig_q = jnp.sum(q * sig_col, axis=-1)
    r_q = jnp.sum(q * proj_r[:, None, :], axis=-1)
    g_q = jnp.sum(q * proj_g[:, None, :], axis=-1)
    b_q = jnp.sum(q * proj_b[:, None, :], axis=-1)

    mask = mask_ref[...]
    sigma = jnp.maximum(sig_q * mask, 0.0)

    dists = (t1 - t0) * dnorm
    alpha = 1.0 - jnp.exp(-sigma * dists)
    alpha_ref[...] = alpha

    logv = jnp.log1p(1e-10 - alpha)
    row = jax.lax.broadcasted_iota(jnp.int32, (N, N), 0)
    col = jax.lax.broadcasted_iota(jnp.int32, (N, N), 1)
    tri = (row < col).astype(f32)
    trans = jnp.exp(jnp.dot(logv, tri, preferred_element_type=f32, precision=jax.lax.Precision.HIGHEST))

    weights = alpha * trans
    abs_light = jnp.where(weights > 1e-4, weights, 0.0)

    rgb_r = jax.nn.sigmoid(r_q)
    rgb_g = jax.nn.sigmoid(g_q)
    rgb_b = jax.nn.sigmoid(b_q)

    acc = jnp.sum(abs_light, axis=-1, keepdims=True)
    out_r = jnp.sum(abs_light * rgb_r, axis=-1, keepdims=True) + (1.0 - acc)
    out_g = jnp.sum(abs_light * rgb_g, axis=-1, keepdims=True) + (1.0 - acc)
    out_b = jnp.sum(abs_light * rgb_b, axis=-1, keepdims=True) + (1.0 - acc)
    depth = jnp.sum(abs_light * t_mid, axis=-1, keepdims=True)

    pack = jnp.concatenate(
        [out_r, out_g, out_b, depth, jnp.zeros((BB, N - 4), f32)], axis=1)
    pack_ref[...] = pack


@jax.jit
def kernel(rays_o, rays_d, t_vals, queries_mask, atoms, W_grid):
    f32 = jnp.float32
    rays_o = rays_o.astype(f32)
    rays_d = rays_d.astype(f32)
    t0 = t_vals[:, :-1].astype(f32)
    t1 = t_vals[:, 1:].astype(f32)
    mask = queries_mask.astype(f32)
    atomsT = atoms.T.astype(f32)
    wg = W_grid.astype(f32)

    grid = (B // BB,)
    alpha, pack = pl.pallas_call(
        _render_block,
        grid=grid,
        in_specs=[
            pl.BlockSpec((BB, 3), lambda i: (i, 0)),
            pl.BlockSpec((BB, 3), lambda i: (i, 0)),
            pl.BlockSpec((BB, N), lambda i: (i, 0)),
            pl.BlockSpec((BB, N), lambda i: (i, 0)),
            pl.BlockSpec((BB, N), lambda i: (i, 0)),
            pl.BlockSpec((49, NUM_ATOMS), lambda i: (0, 0)),
            pl.BlockSpec((3, NUM_ATOMS), lambda i: (0, 0)),
        ],
        out_specs=[
            pl.BlockSpec((BB, N), lambda i: (i, 0)),
            pl.BlockSpec((BB, N), lambda i: (i, 0)),
        ],
        out_shape=[
            jax.ShapeDtypeStruct((B, N), f32),
            jax.ShapeDtypeStruct((B, N), f32),
        ],
    )(rays_o, rays_d, t0, t1, mask, atomsT, wg)

    rgb_out = pack[:, 0:3]
    depth = pack[:, 3]
    return rgb_out, alpha, depth


# speedup vs baseline: 2.5101x; 2.5101x over previous
import functools

import jax
import jax.numpy as jnp
from jax.experimental import pallas as pl

SH_DIM = 16
NUM_ATOMS = 512
B, N = 2048, 128
BB = 8

C0 = 0.28209479177387814
C1 = 0.4886025119029199
C2 = (1.0925484305920792, -1.0925484305920792, 0.31539156525252005,
      -1.0925484305920792, 0.5462742152960396)
C3 = (-0.5900435899266435, 2.890611442640554, -0.4570457994644658,
      0.3731763325901154, -0.4570457994644658, 1.445305721320277,
      -0.5900435899266435)


def _sh_basis(x, y, z):
    xx, yy, zz = x * x, y * y, z * z
    xy, yz, xz = x * y, y * z, x * z
    cols = [
        C0 * jnp.ones_like(x),
        -C1 * y,
        C1 * z,
        -C1 * x,
        C2[0] * xy,
        C2[1] * yz,
        C2[2] * (2.0 * zz - xx - yy),
        C2[3] * xz,
        C2[4] * (xx - yy),
        C3[0] * y * (3.0 * xx - yy),
        C3[1] * xy * z,
        C3[2] * y * (4.0 * zz - xx - yy),
        C3[3] * z * (2.0 * zz - 3.0 * xx - 3.0 * yy),
        C3[4] * x * (4.0 * zz - xx - yy),
        C3[5] * z * (xx - yy),
        C3[6] * x * (xx - 3.0 * yy),
    ]
    return jnp.concatenate(cols, axis=1)


def _render_block(o_ref, d_ref, t0_ref, t1_ref, mask_ref, atomsT_ref,
                  wg_ref, alpha_ref, pack_ref):
    f32 = jnp.float32
    o = o_ref[...]
    d = d_ref[...]
    ox = o[:, 0].reshape(BB, 1)
    oy = o[:, 1].reshape(BB, 1)
    oz = o[:, 2].reshape(BB, 1)
    dx = d[:, 0].reshape(BB, 1)
    dy = d[:, 1].reshape(BB, 1)
    dz = d[:, 2].reshape(BB, 1)

    bf16 = jnp.bfloat16
    wgr = wg_ref[...].astype(bf16).astype(f32)
    w0 = wgr[0, :].reshape(1, NUM_ATOMS, 1)
    w1 = wgr[1, :].reshape(1, NUM_ATOMS, 1)
    w2 = wgr[2, :].reshape(1, NUM_ATOMS, 1)

    t0 = t0_ref[...]
    t1 = t1_ref[...]
    t_mid = 0.5 * (t0 + t1)

    dnorm2 = dx * dx + dy * dy + dz * dz
    dnorm = jnp.sqrt(dnorm2)
    inv_dn = 1.0 / (dnorm + 1e-12)

    sh = _sh_basis(dx * inv_dn, dy * inv_dn, dz * inv_dn)
    atomsT = atomsT_ref[...].astype(bf16).astype(f32)
    proj_r = jnp.dot(sh, atomsT[0:16, :], preferred_element_type=f32, precision=jax.lax.Precision.HIGHEST)
    proj_g = jnp.dot(sh, atomsT[16:32, :], preferred_element_type=f32, precision=jax.lax.Precision.HIGHEST)
    proj_b = jnp.dot(sh, atomsT[32:48, :], preferred_element_type=f32, precision=jax.lax.Precision.HIGHEST)
    sig_col = atomsT[48, :]

    px = (ox + t_mid * dx).astype(bf16).astype(f32)[:, None, :]
    py = (oy + t_mid * dy).astype(bf16).astype(f32)[:, None, :]
    pz = (oz + t_mid * dz).astype(bf16).astype(f32)[:, None, :]
    logits = px * w0 + py * w1 + pz * w2

    m = jnp.max(logits, axis=1, keepdims=True)
    u = jnp.exp(logits - m)
    z = jnp.sum(u, axis=1, keepdims=True)
    q = (u / z).astype(bf16)

    sig_bb = jnp.broadcast_to(sig_col.reshape(1, NUM_ATOMS), (BB, NUM_ATOMS))
    v_all = jnp.stack(
        [proj_r, proj_g, proj_b, sig_bb, sig_bb, sig_bb, sig_bb, sig_bb],
        axis=1).astype(bf16)
    red = jax.lax.dot_general(
        v_all, q, (((2,), (1,)), ((0,), (0,))),
        preferred_element_type=f32)
    r_q = red[:, 0, :]
    g_q = red[:, 1, :]
    b_q = red[:, 2, :]
    sig_q = red[:, 3, :]

    mask = mask_ref[...]
    sigma = jnp.maximum(sig_q * mask, 0.0)

    dists = (t1 - t0) * dnorm
    alpha = 1.0 - jnp.exp(-sigma * dists)
    alpha_ref[...] = alpha

    logv = jnp.log1p(1e-10 - alpha)
    row = jax.lax.broadcasted_iota(jnp.int32, (N, N), 0)
    col = jax.lax.broadcasted_iota(jnp.int32, (N, N), 1)
    tri = (row < col).astype(f32)
    trans = jnp.exp(jnp.dot(logv, tri, preferred_element_type=f32, precision=jax.lax.Precision.HIGHEST))

    weights = alpha * trans
    abs_light = jnp.where(weights > 1e-4, weights, 0.0)

    rgb_r = jax.nn.sigmoid(r_q)
    rgb_g = jax.nn.sigmoid(g_q)
    rgb_b = jax.nn.sigmoid(b_q)

    acc = jnp.sum(abs_light, axis=-1, keepdims=True)
    out_r = jnp.sum(abs_light * rgb_r, axis=-1, keepdims=True) + (1.0 - acc)
    out_g = jnp.sum(abs_light * rgb_g, axis=-1, keepdims=True) + (1.0 - acc)
    out_b = jnp.sum(abs_light * rgb_b, axis=-1, keepdims=True) + (1.0 - acc)
    depth = jnp.sum(abs_light * t_mid, axis=-1, keepdims=True)

    pack = jnp.concatenate(
        [out_r, out_g, out_b, depth, jnp.zeros((BB, N - 4), f32)], axis=1)
    pack_ref[...] = pack


@jax.jit
def kernel(rays_o, rays_d, t_vals, queries_mask, atoms, W_grid):
    f32 = jnp.float32
    rays_o = rays_o.astype(f32)
    rays_d = rays_d.astype(f32)
    t0 = t_vals[:, :-1].astype(f32)
    t1 = t_vals[:, 1:].astype(f32)
    mask = queries_mask.astype(f32)
    atomsT = atoms.T.astype(f32)
    wg = W_grid.astype(f32)

    grid = (B // BB,)
    alpha, pack = pl.pallas_call(
        _render_block,
        grid=grid,
        in_specs=[
            pl.BlockSpec((BB, 3), lambda i: (i, 0)),
            pl.BlockSpec((BB, 3), lambda i: (i, 0)),
            pl.BlockSpec((BB, N), lambda i: (i, 0)),
            pl.BlockSpec((BB, N), lambda i: (i, 0)),
            pl.BlockSpec((BB, N), lambda i: (i, 0)),
            pl.BlockSpec((49, NUM_ATOMS), lambda i: (0, 0)),
            pl.BlockSpec((3, NUM_ATOMS), lambda i: (0, 0)),
        ],
        out_specs=[
            pl.BlockSpec((BB, N), lambda i: (i, 0)),
            pl.BlockSpec((BB, N), lambda i: (i, 0)),
        ],
        out_shape=[
            jax.ShapeDtypeStruct((B, N), f32),
            jax.ShapeDtypeStruct((B, N), f32),
        ],
    )(rays_o, rays_d, t0, t1, mask, atomsT, wg)

    rgb_out = pack[:, 0:3]
    depth = pack[:, 3]
    return rgb_out, alpha, depth


# speedup vs baseline: 5.5603x; 2.2151x over previous
import functools

import jax
import jax.numpy as jnp
from jax.experimental import pallas as pl

SH_DIM = 16
NUM_ATOMS = 512
B, N = 2048, 128
BB = 128

C0 = 0.28209479177387814
C1 = 0.4886025119029199
C2 = (1.0925484305920792, -1.0925484305920792, 0.31539156525252005,
      -1.0925484305920792, 0.5462742152960396)
C3 = (-0.5900435899266435, 2.890611442640554, -0.4570457994644658,
      0.3731763325901154, -0.4570457994644658, 1.445305721320277,
      -0.5900435899266435)


def _sh_basis(x, y, z):
    xx, yy, zz = x * x, y * y, z * z
    xy, yz, xz = x * y, y * z, x * z
    cols = [
        C0 * jnp.ones_like(x),
        -C1 * y,
        C1 * z,
        -C1 * x,
        C2[0] * xy,
        C2[1] * yz,
        C2[2] * (2.0 * zz - xx - yy),
        C2[3] * xz,
        C2[4] * (xx - yy),
        C3[0] * y * (3.0 * xx - yy),
        C3[1] * xy * z,
        C3[2] * y * (4.0 * zz - xx - yy),
        C3[3] * z * (2.0 * zz - 3.0 * xx - 3.0 * yy),
        C3[4] * x * (4.0 * zz - xx - yy),
        C3[5] * z * (xx - yy),
        C3[6] * x * (xx - 3.0 * yy),
    ]
    return jnp.concatenate(cols, axis=1)


def _render_block(o_ref, d_ref, t0_ref, t1_ref, mask_ref, atomsT_ref,
                  wg_ref, alpha_ref, pack_ref):
    f32 = jnp.float32
    o = o_ref[...]
    d = d_ref[...]
    ox = o[:, 0].reshape(BB, 1)
    oy = o[:, 1].reshape(BB, 1)
    oz = o[:, 2].reshape(BB, 1)
    dx = d[:, 0].reshape(BB, 1)
    dy = d[:, 1].reshape(BB, 1)
    dz = d[:, 2].reshape(BB, 1)

    bf16 = jnp.bfloat16
    wgT_bf = wg_ref[...].T.astype(bf16)

    t0 = t0_ref[...]
    t1 = t1_ref[...]
    t_mid = 0.5 * (t0 + t1)

    dnorm2 = dx * dx + dy * dy + dz * dz
    dnorm = jnp.sqrt(dnorm2)
    inv_dn = 1.0 / (dnorm + 1e-12)

    sh = _sh_basis(dx * inv_dn, dy * inv_dn, dz * inv_dn)
    atomsT = atomsT_ref[...].astype(bf16).astype(f32)
    proj_r = jnp.dot(sh, atomsT[0:16, :], preferred_element_type=f32, precision=jax.lax.Precision.HIGHEST)
    proj_g = jnp.dot(sh, atomsT[16:32, :], preferred_element_type=f32, precision=jax.lax.Precision.HIGHEST)
    proj_b = jnp.dot(sh, atomsT[32:48, :], preferred_element_type=f32, precision=jax.lax.Precision.HIGHEST)
    sig_col = atomsT[48, :]

    p3 = jnp.stack([ox + t_mid * dx, oy + t_mid * dy, oz + t_mid * dz],
                   axis=1).astype(bf16)
    wgT_all = jnp.broadcast_to(wgT_bf[None], (BB, NUM_ATOMS, 3))
    logits = jax.lax.dot_general(
        wgT_all, p3, (((2,), (1,)), ((0,), (0,))),
        preferred_element_type=f32)

    m = jnp.max(logits, axis=1, keepdims=True)
    u_bf = jnp.exp(logits - m).astype(bf16)

    sig_bb = jnp.broadcast_to(sig_col.reshape(1, NUM_ATOMS), (BB, NUM_ATOMS))
    one_bb = jnp.ones((BB, NUM_ATOMS), f32)
    v_all = jnp.stack(
        [proj_r, proj_g, proj_b, sig_bb, one_bb, one_bb, one_bb, one_bb],
        axis=1).astype(bf16)
    red = jax.lax.dot_general(
        v_all, u_bf, (((2,), (1,)), ((0,), (0,))),
        preferred_element_type=f32)
    rz = 1.0 / red[:, 4, :]
    r_q = red[:, 0, :] * rz
    g_q = red[:, 1, :] * rz
    b_q = red[:, 2, :] * rz
    sig_q = red[:, 3, :] * rz

    mask = mask_ref[...]
    sigma = jnp.maximum(sig_q * mask, 0.0)

    dists = (t1 - t0) * dnorm
    alpha = 1.0 - jnp.exp(-sigma * dists)
    alpha_ref[...] = alpha

    logv = jnp.log1p(1e-10 - alpha)
    row = jax.lax.broadcasted_iota(jnp.int32, (N, N), 0)
    col = jax.lax.broadcasted_iota(jnp.int32, (N, N), 1)
    tri = (row < col).astype(f32)
    trans = jnp.exp(jnp.dot(logv, tri, preferred_element_type=f32, precision=jax.lax.Precision.HIGHEST))

    weights = alpha * trans
    abs_light = jnp.where(weights > 1e-4, weights, 0.0)

    rgb_r = jax.nn.sigmoid(r_q)
    rgb_g = jax.nn.sigmoid(g_q)
    rgb_b = jax.nn.sigmoid(b_q)

    acc = jnp.sum(abs_light, axis=-1, keepdims=True)
    out_r = jnp.sum(abs_light * rgb_r, axis=-1, keepdims=True) + (1.0 - acc)
    out_g = jnp.sum(abs_light * rgb_g, axis=-1, keepdims=True) + (1.0 - acc)
    out_b = jnp.sum(abs_light * rgb_b, axis=-1, keepdims=True) + (1.0 - acc)
    depth = jnp.sum(abs_light * t_mid, axis=-1, keepdims=True)

    pack = jnp.concatenate(
        [out_r, out_g, out_b, depth, jnp.zeros((BB, N - 4), f32)], axis=1)
    pack_ref[...] = pack


@jax.jit
def kernel(rays_o, rays_d, t_vals, queries_mask, atoms, W_grid):
    f32 = jnp.float32
    rays_o = rays_o.astype(f32)
    rays_d = rays_d.astype(f32)
    t0 = t_vals[:, :-1].astype(f32)
    t1 = t_vals[:, 1:].astype(f32)
    mask = queries_mask.astype(f32)
    atomsT = atoms.T.astype(f32)
    wg = W_grid.astype(f32)

    grid = (B // BB,)
    alpha, pack = pl.pallas_call(
        _render_block,
        grid=grid,
        in_specs=[
            pl.BlockSpec((BB, 3), lambda i: (i, 0)),
            pl.BlockSpec((BB, 3), lambda i: (i, 0)),
            pl.BlockSpec((BB, N), lambda i: (i, 0)),
            pl.BlockSpec((BB, N), lambda i: (i, 0)),
            pl.BlockSpec((BB, N), lambda i: (i, 0)),
            pl.BlockSpec((49, NUM_ATOMS), lambda i: (0, 0)),
            pl.BlockSpec((3, NUM_ATOMS), lambda i: (0, 0)),
        ],
        out_specs=[
            pl.BlockSpec((BB, N), lambda i: (i, 0)),
            pl.BlockSpec((BB, N), lambda i: (i, 0)),
        ],
        out_shape=[
            jax.ShapeDtypeStruct((B, N), f32),
            jax.ShapeDtypeStruct((B, N), f32),
        ],
    )(rays_o, rays_d, t0, t1, mask, atomsT, wg)

    rgb_out = pack[:, 0:3]
    depth = pack[:, 3]
    return rgb_out, alpha, depth


# speedup vs baseline: 5.9238x; 1.0654x over previous
import functools

import jax
import jax.numpy as jnp
from jax.experimental import pallas as pl

SH_DIM = 16
NUM_ATOMS = 512
B, N = 2048, 128
BB = 128

C0 = 0.28209479177387814
C1 = 0.4886025119029199
C2 = (1.0925484305920792, -1.0925484305920792, 0.31539156525252005,
      -1.0925484305920792, 0.5462742152960396)
C3 = (-0.5900435899266435, 2.890611442640554, -0.4570457994644658,
      0.3731763325901154, -0.4570457994644658, 1.445305721320277,
      -0.5900435899266435)


def _sh_basis(x, y, z):
    xx, yy, zz = x * x, y * y, z * z
    xy, yz, xz = x * y, y * z, x * z
    cols = [
        C0 * jnp.ones_like(x),
        -C1 * y,
        C1 * z,
        -C1 * x,
        C2[0] * xy,
        C2[1] * yz,
        C2[2] * (2.0 * zz - xx - yy),
        C2[3] * xz,
        C2[4] * (xx - yy),
        C3[0] * y * (3.0 * xx - yy),
        C3[1] * xy * z,
        C3[2] * y * (4.0 * zz - xx - yy),
        C3[3] * z * (2.0 * zz - 3.0 * xx - 3.0 * yy),
        C3[4] * x * (4.0 * zz - xx - yy),
        C3[5] * z * (xx - yy),
        C3[6] * x * (xx - 3.0 * yy),
    ]
    return jnp.concatenate(cols, axis=1)


def _render_block(o_ref, d_ref, t0_ref, t1_ref, mask_ref, atomsT_ref,
                  wg_ref, alpha_ref, pack_ref):
    f32 = jnp.float32
    o = o_ref[...]
    d = d_ref[...]
    ox = o[:, 0].reshape(BB, 1)
    oy = o[:, 1].reshape(BB, 1)
    oz = o[:, 2].reshape(BB, 1)
    dx = d[:, 0].reshape(BB, 1)
    dy = d[:, 1].reshape(BB, 1)
    dz = d[:, 2].reshape(BB, 1)

    bf16 = jnp.bfloat16
    wgT_bf = wg_ref[...].T.astype(bf16)

    t0 = t0_ref[...]
    t1 = t1_ref[...]
    t_mid = 0.5 * (t0 + t1)

    dnorm2 = dx * dx + dy * dy + dz * dz
    dnorm = jnp.sqrt(dnorm2)
    inv_dn = 1.0 / (dnorm + 1e-12)

    sh = _sh_basis(dx * inv_dn, dy * inv_dn, dz * inv_dn)
    atomsT = atomsT_ref[...].astype(bf16).astype(f32)
    proj_r = jnp.dot(sh, atomsT[0:16, :], preferred_element_type=f32)
    proj_g = jnp.dot(sh, atomsT[16:32, :], preferred_element_type=f32)
    proj_b = jnp.dot(sh, atomsT[32:48, :], preferred_element_type=f32)
    sig_col = atomsT[48, :]

    p3 = jnp.stack([ox + t_mid * dx, oy + t_mid * dy, oz + t_mid * dz],
                   axis=1).astype(bf16)
    wgT_all = jnp.broadcast_to(wgT_bf[None], (BB, NUM_ATOMS, 3))
    logits = jax.lax.dot_general(
        wgT_all, p3, (((2,), (1,)), ((0,), (0,))),
        preferred_element_type=f32)

    m = jnp.max(logits, axis=1, keepdims=True)
    u_bf = jnp.exp(logits - m).astype(bf16)

    proj3 = jnp.stack([proj_r, proj_g, proj_b], axis=1).astype(bf16)
    const5 = jnp.concatenate(
        [sig_col.reshape(1, NUM_ATOMS),
         jnp.ones((4, NUM_ATOMS), f32)], axis=0).astype(bf16)
    v_all = jnp.concatenate(
        [proj3, jnp.broadcast_to(const5[None], (BB, 5, NUM_ATOMS))],
        axis=1)
    red = jax.lax.dot_general(
        v_all, u_bf, (((2,), (1,)), ((0,), (0,))),
        preferred_element_type=f32)
    rz = 1.0 / red[:, 4, :]
    r_q = red[:, 0, :] * rz
    g_q = red[:, 1, :] * rz
    b_q = red[:, 2, :] * rz
    sig_q = red[:, 3, :] * rz

    mask = mask_ref[...]
    sigma = jnp.maximum(sig_q * mask, 0.0)

    dists = (t1 - t0) * dnorm
    alpha = 1.0 - jnp.exp(-sigma * dists)
    alpha_ref[...] = alpha

    logv = jnp.log1p(1e-10 - alpha)
    row = jax.lax.broadcasted_iota(jnp.int32, (N, N), 0)
    col = jax.lax.broadcasted_iota(jnp.int32, (N, N), 1)
    tri = (row < col).astype(f32)
    trans = jnp.exp(jnp.dot(logv, tri, preferred_element_type=f32, precision=jax.lax.Precision.HIGHEST))

    weights = alpha * trans
    abs_light = jnp.where(weights > 1e-4, weights, 0.0)

    rgb_r = jax.nn.sigmoid(r_q)
    rgb_g = jax.nn.sigmoid(g_q)
    rgb_b = jax.nn.sigmoid(b_q)

    acc = jnp.sum(abs_light, axis=-1, keepdims=True)
    out_r = jnp.sum(abs_light * rgb_r, axis=-1, keepdims=True) + (1.0 - acc)
    out_g = jnp.sum(abs_light * rgb_g, axis=-1, keepdims=True) + (1.0 - acc)
    out_b = jnp.sum(abs_light * rgb_b, axis=-1, keepdims=True) + (1.0 - acc)
    depth = jnp.sum(abs_light * t_mid, axis=-1, keepdims=True)

    pack = jnp.concatenate(
        [out_r, out_g, out_b, depth, jnp.zeros((BB, N - 4), f32)], axis=1)
    pack_ref[...] = pack


@jax.jit
def kernel(rays_o, rays_d, t_vals, queries_mask, atoms, W_grid):
    f32 = jnp.float32
    rays_o = rays_o.astype(f32)
    rays_d = rays_d.astype(f32)
    t0 = t_vals[:, :-1].astype(f32)
    t1 = t_vals[:, 1:].astype(f32)
    mask = queries_mask.astype(f32)
    atomsT = atoms.T.astype(f32)
    wg = W_grid.astype(f32)

    grid = (B // BB,)
    alpha, pack = pl.pallas_call(
        _render_block,
        grid=grid,
        in_specs=[
            pl.BlockSpec((BB, 3), lambda i: (i, 0)),
            pl.BlockSpec((BB, 3), lambda i: (i, 0)),
            pl.BlockSpec((BB, N), lambda i: (i, 0)),
            pl.BlockSpec((BB, N), lambda i: (i, 0)),
            pl.BlockSpec((BB, N), lambda i: (i, 0)),
            pl.BlockSpec((49, NUM_ATOMS), lambda i: (0, 0)),
            pl.BlockSpec((3, NUM_ATOMS), lambda i: (0, 0)),
        ],
        out_specs=[
            pl.BlockSpec((BB, N), lambda i: (i, 0)),
            pl.BlockSpec((BB, N), lambda i: (i, 0)),
        ],
        out_shape=[
            jax.ShapeDtypeStruct((B, N), f32),
            jax.ShapeDtypeStruct((B, N), f32),
        ],
    )(rays_o, rays_d, t0, t1, mask, atomsT, wg)

    rgb_out = pack[:, 0:3]
    depth = pack[:, 3]
    return rgb_out, alpha, depth
